# baseline (speedup 1.0000x reference)
import functools

import jax
import jax.numpy as jnp
from jax import lax
from jax.experimental import pallas as pl
from jax.experimental.pallas import tpu as pltpu
from jax.experimental.pallas import tpu_sc as plsc

NUM_EMBEDDINGS = 1024
EMBEDDING_DIM = 64
COMMITMENT_COST = 0.25

_BLOCK_ROWS = 512
_N_TOKENS = 16 * 576


def _argmin_kernel(x_ref, cbp_ref, lam_ref, a2_ref,
                   idx_ref, dcb2_ref, loss_ref):
    i = pl.program_id(0)
    x = x_ref[...]
    cb = cbp_ref[:, 0:EMBEDDING_DIM]
    cbn = cbp_ref[:, EMBEDDING_DIM:2 * EMBEDDING_DIM]
    lam = lam_ref[...]

    dcb = (1.0 - lam) * cb + lam * cbn
    b2 = jnp.sum(dcb * dcb, axis=1)
    col = jax.lax.broadcasted_iota(jnp.int32, (1, NUM_EMBEDDINGS), 1)
    b2 = jnp.where(col[0] == NUM_EMBEDDINGS - 1, jnp.float32(1e30), b2)

    @pl.when(i == 0)
    def _():
        dcb2_ref[...] = jnp.concatenate([dcb, dcb], axis=1)

    a2 = a2_ref[...]
    m = jax.lax.dot_general(
        x, dcb, (((1,), (1,)), ((), ())),
        preferred_element_type=jnp.float32)
    scores = jnp.sqrt(jnp.maximum((a2 + b2[None, :]) - 2.0 * m, 0.0))

    mn = jnp.min(scores, axis=1, keepdims=True)
    cols = jax.lax.broadcasted_iota(jnp.int32, scores.shape, 1)
    idx_ref[...] = jnp.min(jnp.where(scores == mn, cols, NUM_EMBEDDINGS),
                           axis=1, keepdims=True)

    part = (jnp.sum(mn * mn) * jnp.float32(
        (1.0 + COMMITMENT_COST) / (_N_TOKENS * EMBEDDING_DIM))).reshape(1, 1)

    @pl.when(i == 0)
    def _():
        loss_ref[...] = part

    @pl.when(i != 0)
    def _():
        loss_ref[...] += part


def _make_sc_gather(n, b_per_w, nc, ns):
    mesh = plsc.VectorSubcoreMesh(core_axis_name="c", subcore_axis_name="s")

    @functools.partial(
        pl.kernel, mesh=mesh,
        out_type=jax.ShapeDtypeStruct((n, 2 * EMBEDDING_DIM), jnp.float32),
        scratch_types=[
            pltpu.VMEM((b_per_w,), jnp.int32),
            pltpu.VMEM((b_per_w, 2 * EMBEDDING_DIM), jnp.float32),
            pltpu.SemaphoreType.DMA,
        ],
    )
    def sc_gather(table_hbm, idx_hbm, out_hbm, idx_v, rows_v, sem):
        wid = lax.axis_index("s") * nc + lax.axis_index("c")
        base = wid * b_per_w
        pltpu.sync_copy(idx_hbm.at[pl.ds(base, b_per_w)], idx_v)
        pltpu.async_copy(table_hbm.at[idx_v], rows_v, sem).wait()
        pltpu.sync_copy(rows_v, out_hbm.at[pl.ds(base, b_per_w)])

    return sc_gather


@jax.jit
def kernel(z, lambda_pairs, codebook):
    n = z.shape[0] * z.shape[1]
    flat = z.reshape(n, EMBEDDING_DIM)
    cb_next = jnp.concatenate([codebook[1:], codebook[:1]], axis=0)
    lam_pad = jnp.concatenate(
        [lambda_pairs, jnp.zeros((1, 1), jnp.float32)], axis=0)
    cbp = jnp.concatenate([codebook, cb_next], axis=1)
    a2 = jnp.sum(flat ** 2, axis=1, keepdims=True)

    grid = n // _BLOCK_ROWS
    idx, dcb2, loss = pl.pallas_call(
        _argmin_kernel,
        grid=(grid,),
        in_specs=[
            pl.BlockSpec((_BLOCK_ROWS, EMBEDDING_DIM), lambda i: (i, 0)),
            pl.BlockSpec((NUM_EMBEDDINGS, 2 * EMBEDDING_DIM),
                         lambda i: (0, 0)),
            pl.BlockSpec((NUM_EMBEDDINGS, 1), lambda i: (0, 0)),
            pl.BlockSpec((_BLOCK_ROWS, 1), lambda i: (i, 0)),
        ],
        out_specs=[
            pl.BlockSpec((_BLOCK_ROWS, 1), lambda i: (i, 0)),
            pl.BlockSpec((NUM_EMBEDDINGS, 2 * EMBEDDING_DIM),
                         lambda i: (0, 0)),
            pl.BlockSpec((1, 1), lambda i: (0, 0)),
        ],
        out_shape=[
            jax.ShapeDtypeStruct((n, 1), jnp.int32),
            jax.ShapeDtypeStruct((NUM_EMBEDDINGS, 2 * EMBEDDING_DIM),
                                 jnp.float32),
            jax.ShapeDtypeStruct((1, 1), jnp.float32),
        ],
    )(flat, cbp, lam_pad, a2)

    info = plsc.get_sparse_core_info()
    nw = info.num_cores * info.num_subcores
    g = _make_sc_gather(n, n // nw, info.num_cores, info.num_subcores)(
        dcb2, idx.reshape(n))

    return (g[:, 0:EMBEDDING_DIM].reshape(z.shape), loss[0, 0],
            idx[:, 0].reshape(z.shape[:-1]))


# speedup vs baseline: 1.1524x; 1.1524x over previous
import functools

import jax
import jax.numpy as jnp
from jax import lax
from jax.experimental import pallas as pl
from jax.experimental.pallas import tpu as pltpu
from jax.experimental.pallas import tpu_sc as plsc

NUM_EMBEDDINGS = 1024
EMBEDDING_DIM = 64
COMMITMENT_COST = 0.25

_BLOCK_ROWS = 2304
_N_TOKENS = 16 * 576


def _argmin_kernel(x_ref, cbp_ref, lam_ref, a2_ref,
                   idx_ref, dcb2_ref, loss_ref):
    i = pl.program_id(0)
    x = x_ref[...]
    cb = cbp_ref[:, 0:EMBEDDING_DIM]
    cbn = cbp_ref[:, EMBEDDING_DIM:2 * EMBEDDING_DIM]
    lam = lam_ref[...]

    dcb = (1.0 - lam) * cb + lam * cbn
    b2 = jnp.sum(dcb * dcb, axis=1)
    col = jax.lax.broadcasted_iota(jnp.int32, (1, NUM_EMBEDDINGS), 1)
    b2 = jnp.where(col[0] == NUM_EMBEDDINGS - 1, jnp.float32(1e30), b2)

    @pl.when(i == 0)
    def _():
        dcb2_ref[...] = jnp.concatenate([dcb, dcb], axis=1)

    a2 = a2_ref[...]
    m = jax.lax.dot_general(
        x, dcb, (((1,), (1,)), ((), ())),
        preferred_element_type=jnp.float32)
    scores = jnp.sqrt(jnp.maximum((a2 + b2[None, :]) - 2.0 * m, 0.0))

    mn = jnp.min(scores, axis=1, keepdims=True)
    cols = jax.lax.broadcasted_iota(jnp.int32, scores.shape, 1)
    idx_ref[...] = jnp.min(jnp.where(scores == mn, cols, NUM_EMBEDDINGS),
                           axis=1, keepdims=True)

    part = (jnp.sum(mn * mn) * jnp.float32(
        (1.0 + COMMITMENT_COST) / (_N_TOKENS * EMBEDDING_DIM))).reshape(1, 1)

    @pl.when(i == 0)
    def _():
        loss_ref[...] = part

    @pl.when(i != 0)
    def _():
        loss_ref[...] += part


def _make_sc_gather(n, b_per_w, nc, ns):
    mesh = plsc.VectorSubcoreMesh(core_axis_name="c", subcore_axis_name="s")

    @functools.partial(
        pl.kernel, mesh=mesh,
        out_type=jax.ShapeDtypeStruct((n, 2 * EMBEDDING_DIM), jnp.float32),
        scratch_types=[
            pltpu.VMEM((b_per_w,), jnp.int32),
            pltpu.VMEM((b_per_w, 2 * EMBEDDING_DIM), jnp.float32),
            pltpu.SemaphoreType.DMA,
        ],
    )
    def sc_gather(table_hbm, idx_hbm, out_hbm, idx_v, rows_v, sem):
        wid = lax.axis_index("s") * nc + lax.axis_index("c")
        base = wid * b_per_w
        pltpu.sync_copy(idx_hbm.at[pl.ds(base, b_per_w)], idx_v)
        pltpu.async_copy(table_hbm.at[idx_v], rows_v, sem).wait()
        pltpu.sync_copy(rows_v, out_hbm.at[pl.ds(base, b_per_w)])

    return sc_gather


@jax.jit
def kernel(z, lambda_pairs, codebook):
    n = z.shape[0] * z.shape[1]
    flat = z.reshape(n, EMBEDDING_DIM)
    cb_next = jnp.concatenate([codebook[1:], codebook[:1]], axis=0)
    lam_pad = jnp.concatenate(
        [lambda_pairs, jnp.zeros((1, 1), jnp.float32)], axis=0)
    cbp = jnp.concatenate([codebook, cb_next], axis=1)
    a2 = jnp.sum(flat ** 2, axis=1, keepdims=True)

    grid = n // _BLOCK_ROWS
    idx, dcb2, loss = pl.pallas_call(
        _argmin_kernel,
        grid=(grid,),
        in_specs=[
            pl.BlockSpec((_BLOCK_ROWS, EMBEDDING_DIM), lambda i: (i, 0)),
            pl.BlockSpec((NUM_EMBEDDINGS, 2 * EMBEDDING_DIM),
                         lambda i: (0, 0)),
            pl.BlockSpec((NUM_EMBEDDINGS, 1), lambda i: (0, 0)),
            pl.BlockSpec((_BLOCK_ROWS, 1), lambda i: (i, 0)),
        ],
        out_specs=[
            pl.BlockSpec((_BLOCK_ROWS, 1), lambda i: (i, 0)),
            pl.BlockSpec((NUM_EMBEDDINGS, 2 * EMBEDDING_DIM),
                         lambda i: (0, 0)),
            pl.BlockSpec((1, 1), lambda i: (0, 0)),
        ],
        out_shape=[
            jax.ShapeDtypeStruct((n, 1), jnp.int32),
            jax.ShapeDtypeStruct((NUM_EMBEDDINGS, 2 * EMBEDDING_DIM),
                                 jnp.float32),
            jax.ShapeDtypeStruct((1, 1), jnp.float32),
        ],
    )(flat, cbp, lam_pad, a2)

    info = plsc.get_sparse_core_info()
    nw = info.num_cores * info.num_subcores
    g = _make_sc_gather(n, n // nw, info.num_cores, info.num_subcores)(
        dcb2, idx.reshape(n))

    return (g[:, 0:EMBEDDING_DIM].reshape(z.shape), loss[0, 0],
            idx[:, 0].reshape(z.shape[:-1]))


# speedup vs baseline: 1.1590x; 1.0058x over previous
import functools

import jax
import jax.numpy as jnp
from jax import lax
from jax.experimental import pallas as pl
from jax.experimental.pallas import tpu as pltpu
from jax.experimental.pallas import tpu_sc as plsc

NUM_EMBEDDINGS = 1024
EMBEDDING_DIM = 64
COMMITMENT_COST = 0.25

_BLOCK_ROWS = 4608
_N_TOKENS = 16 * 576


def _argmin_kernel(x_ref, cbp_ref, lam_ref, a2_ref,
                   idx_ref, dcb2_ref, loss_ref):
    i = pl.program_id(0)
    x = x_ref[...]
    cb = cbp_ref[:, 0:EMBEDDING_DIM]
    cbn = cbp_ref[:, EMBEDDING_DIM:2 * EMBEDDING_DIM]
    lam = lam_ref[...]

    dcb = (1.0 - lam) * cb + lam * cbn
    b2 = jnp.sum(dcb * dcb, axis=1)
    col = jax.lax.broadcasted_iota(jnp.int32, (1, NUM_EMBEDDINGS), 1)
    b2 = jnp.where(col[0] == NUM_EMBEDDINGS - 1, jnp.float32(1e30), b2)

    @pl.when(i == 0)
    def _():
        dcb2_ref[...] = jnp.concatenate([dcb, dcb], axis=1)

    a2 = a2_ref[...]
    m = jax.lax.dot_general(
        x, dcb, (((1,), (1,)), ((), ())),
        preferred_element_type=jnp.float32)
    scores = jnp.sqrt(jnp.maximum((a2 + b2[None, :]) - 2.0 * m, 0.0))

    mn = jnp.min(scores, axis=1, keepdims=True)
    cols = jax.lax.broadcasted_iota(jnp.int32, scores.shape, 1)
    idx_ref[...] = jnp.min(jnp.where(scores == mn, cols, NUM_EMBEDDINGS),
                           axis=1, keepdims=True)

    part = (jnp.sum(mn * mn) * jnp.float32(
        (1.0 + COMMITMENT_COST) / (_N_TOKENS * EMBEDDING_DIM))).reshape(1, 1)

    @pl.when(i == 0)
    def _():
        loss_ref[...] = part

    @pl.when(i != 0)
    def _():
        loss_ref[...] += part


def _make_sc_gather(n, b_per_w, nc, ns):
    mesh = plsc.VectorSubcoreMesh(core_axis_name="c", subcore_axis_name="s")

    @functools.partial(
        pl.kernel, mesh=mesh,
        out_type=jax.ShapeDtypeStruct((n, 2 * EMBEDDING_DIM), jnp.float32),
        scratch_types=[
            pltpu.VMEM((b_per_w,), jnp.int32),
            pltpu.VMEM((b_per_w, 2 * EMBEDDING_DIM), jnp.float32),
            pltpu.SemaphoreType.DMA,
        ],
    )
    def sc_gather(table_hbm, idx_hbm, out_hbm, idx_v, rows_v, sem):
        wid = lax.axis_index("s") * nc + lax.axis_index("c")
        base = wid * b_per_w
        pltpu.sync_copy(idx_hbm.at[pl.ds(base, b_per_w)], idx_v)
        pltpu.async_copy(table_hbm.at[idx_v], rows_v, sem).wait()
        pltpu.sync_copy(rows_v, out_hbm.at[pl.ds(base, b_per_w)])

    return sc_gather


@jax.jit
def kernel(z, lambda_pairs, codebook):
    n = z.shape[0] * z.shape[1]
    flat = z.reshape(n, EMBEDDING_DIM)
    cb_next = jnp.concatenate([codebook[1:], codebook[:1]], axis=0)
    lam_pad = jnp.concatenate(
        [lambda_pairs, jnp.zeros((1, 1), jnp.float32)], axis=0)
    cbp = jnp.concatenate([codebook, cb_next], axis=1)
    a2 = jnp.sum(flat ** 2, axis=1, keepdims=True)

    grid = n // _BLOCK_ROWS
    idx, dcb2, loss = pl.pallas_call(
        _argmin_kernel,
        grid=(grid,),
        in_specs=[
            pl.BlockSpec((_BLOCK_ROWS, EMBEDDING_DIM), lambda i: (i, 0)),
            pl.BlockSpec((NUM_EMBEDDINGS, 2 * EMBEDDING_DIM),
                         lambda i: (0, 0)),
            pl.BlockSpec((NUM_EMBEDDINGS, 1), lambda i: (0, 0)),
            pl.BlockSpec((_BLOCK_ROWS, 1), lambda i: (i, 0)),
        ],
        out_specs=[
            pl.BlockSpec((_BLOCK_ROWS, 1), lambda i: (i, 0)),
            pl.BlockSpec((NUM_EMBEDDINGS, 2 * EMBEDDING_DIM),
                         lambda i: (0, 0)),
            pl.BlockSpec((1, 1), lambda i: (0, 0)),
        ],
        out_shape=[
            jax.ShapeDtypeStruct((n, 1), jnp.int32),
            jax.ShapeDtypeStruct((NUM_EMBEDDINGS, 2 * EMBEDDING_DIM),
                                 jnp.float32),
            jax.ShapeDtypeStruct((1, 1), jnp.float32),
        ],
    )(flat, cbp, lam_pad, a2)

    info = plsc.get_sparse_core_info()
    nw = info.num_cores * info.num_subcores
    g = _make_sc_gather(n, n // nw, info.num_cores, info.num_subcores)(
        dcb2, idx.reshape(n))

    return (g[:, 0:EMBEDDING_DIM].reshape(z.shape), loss[0, 0],
            idx[:, 0].reshape(z.shape[:-1]))
